# baseline (speedup 1.0000x reference)
import jax
import jax.numpy as jnp
from jax.experimental import pallas as pl

_LAMBDA = 0.8
_K = 8


def _norm_rows(x):
    n = jnp.sqrt(jnp.sum(x * x, axis=-1, keepdims=True))
    return x / jnp.maximum(n, 1e-12)


def _bitonic8_desc(x):
    y = [None] * 8
    for i in range(4):
        y[i] = jnp.maximum(x[i], x[i + 4])
        y[i + 4] = jnp.minimum(x[i], x[i + 4])
    z = [None] * 8
    for h in (0, 4):
        for i in range(2):
            z[h + i] = jnp.maximum(y[h + i], y[h + i + 2])
            z[h + i + 2] = jnp.minimum(y[h + i], y[h + i + 2])
    w = [None] * 8
    for i in (0, 2, 4, 6):
        w[i] = jnp.maximum(z[i], z[i + 1])
        w[i + 1] = jnp.minimum(z[i], z[i + 1])
    return w


def _top8_threshold(scores, n):
    tiles = n // 128
    t = [scores[:, 128 * i:128 * (i + 1)] for i in range(tiles)]
    s2 = [(jnp.maximum(t[2 * i], t[2 * i + 1]),
           jnp.minimum(t[2 * i], t[2 * i + 1])) for i in range(tiles // 2)]
    s4 = []
    for i in range(tiles // 4):
        (a1, a2), (b1, b2) = s2[2 * i], s2[2 * i + 1]
        c1 = jnp.maximum(a1, b1)
        t1 = jnp.minimum(a1, b1)
        c4 = jnp.minimum(a2, b2)
        t2 = jnp.maximum(a2, b2)
        s4.append((c1, jnp.maximum(t1, t2), jnp.minimum(t1, t2), c4))
    s8 = []
    for i in range(tiles // 8):
        a, b = s4[2 * i], s4[2 * i + 1]
        s8.append(_bitonic8_desc(list(a) + list(b)[::-1]))
    while len(s8) > 1:
        a, b = s8[0], s8[1]
        top = [jnp.maximum(a[i], b[7 - i]) for i in range(8)]
        s8 = [_bitonic8_desc(top)] + s8[2:]
    c = list(s8[0])
    neg_inf = jnp.float32(-jnp.inf)
    for _ in range(_K - 1):
        mx = jnp.max(c[0], axis=-1, keepdims=True)
        popped = c[0] == mx
        for i in range(7):
            c[i] = jnp.where(popped, c[i + 1], c[i])
        c[7] = jnp.where(popped, neg_inf, c[7])
    return jnp.max(c[0], axis=-1, keepdims=True)


def _gam_kernel(x_ref, o_ref, *, rb):
    fb = _norm_rows(x_ref[0])
    j = pl.program_id(1)
    a = _norm_rows(x_ref[0, pl.ds(j * rb, rb), :])

    scores = jax.lax.dot_general(
        a * jnp.float32(1.4426950408889634), fb,
        (((1,), (1,)), ((), ())), preferred_element_type=jnp.float32)

    m = _top8_threshold(scores, scores.shape[-1])

    p = jnp.where(scores >= m, jnp.exp2(scores), 0.0)
    att = jax.lax.dot_general(
        p, fb, (((1,), (0,)), ((), ())), preferred_element_type=jnp.float32)
    att = att * (1.0 / jnp.sum(p, axis=-1, keepdims=True))
    o_ref[0] = _norm_rows(a * _LAMBDA + att * (1.0 - _LAMBDA))


@jax.jit
def kernel(feats, node):
    del node
    import functools
    b, n, d = feats.shape
    rb = 1024
    out = pl.pallas_call(
        functools.partial(_gam_kernel, rb=rb),
        grid=(b, n // rb),
        in_specs=[pl.BlockSpec((1, n, d), lambda i, j: (i, 0, 0))],
        out_specs=pl.BlockSpec((1, rb, d), lambda i, j: (i, j, 0)),
        out_shape=jax.ShapeDtypeStruct((b, n, d), jnp.float32),
    )(feats)
    return out


# speedup vs baseline: 1.0066x; 1.0066x over previous
import functools

import jax
import jax.numpy as jnp
from jax.experimental import pallas as pl

_LAMBDA = 0.8
_K = 8


def _norm_rows(x):
    n = jnp.sqrt(jnp.sum(x * x, axis=-1, keepdims=True))
    return x / jnp.maximum(n, 1e-12)


def _bitonic8_desc(x):
    y = [None] * 8
    for i in range(4):
        y[i] = jnp.maximum(x[i], x[i + 4])
        y[i + 4] = jnp.minimum(x[i], x[i + 4])
    z = [None] * 8
    for h in (0, 4):
        for i in range(2):
            z[h + i] = jnp.maximum(y[h + i], y[h + i + 2])
            z[h + i + 2] = jnp.minimum(y[h + i], y[h + i + 2])
    w = [None] * 8
    for i in (0, 2, 4, 6):
        w[i] = jnp.maximum(z[i], z[i + 1])
        w[i + 1] = jnp.minimum(z[i], z[i + 1])
    return w


def _top8_threshold(scores, n):
    tiles = n // 128
    t = [scores[:, 128 * i:128 * (i + 1)] for i in range(tiles)]
    s2 = [(jnp.maximum(t[2 * i], t[2 * i + 1]),
           jnp.minimum(t[2 * i], t[2 * i + 1])) for i in range(tiles // 2)]
    s4 = []
    for i in range(tiles // 4):
        (a1, a2), (b1, b2) = s2[2 * i], s2[2 * i + 1]
        c1 = jnp.maximum(a1, b1)
        t1 = jnp.minimum(a1, b1)
        c4 = jnp.minimum(a2, b2)
        t2 = jnp.maximum(a2, b2)
        s4.append((c1, jnp.maximum(t1, t2), jnp.minimum(t1, t2), c4))
    s8 = []
    for i in range(tiles // 8):
        a, b = s4[2 * i], s4[2 * i + 1]
        s8.append(_bitonic8_desc(list(a) + list(b)[::-1]))
    while len(s8) > 1:
        a, b = s8[0], s8[1]
        top = [jnp.maximum(a[i], b[7 - i]) for i in range(8)]
        s8 = [_bitonic8_desc(top)] + s8[2:]
    c = list(s8[0])
    neg_inf = jnp.float32(-jnp.inf)
    for _ in range(_K - 1):
        mx = jnp.max(c[0], axis=-1, keepdims=True)
        popped = c[0] == mx
        for i in range(7):
            c[i] = jnp.where(popped, c[i + 1], c[i])
        c[7] = jnp.where(popped, neg_inf, c[7])
    return jnp.max(c[0], axis=-1, keepdims=True)


def _gam_kernel(x_ref, o_ref, *, rb):
    fb = _norm_rows(x_ref[0])
    j = pl.program_id(1)
    a = _norm_rows(x_ref[0, pl.ds(j * rb, rb), :])

    scores = jax.lax.dot_general(
        a, fb, (((1,), (1,)), ((), ())), preferred_element_type=jnp.float32)

    m = _top8_threshold(scores, scores.shape[-1])

    p = jnp.where(scores >= m, jnp.exp(scores), 0.0)
    att = jax.lax.dot_general(
        p, fb, (((1,), (0,)), ((), ())), preferred_element_type=jnp.float32)
    att = att * (1.0 / jnp.sum(p, axis=-1, keepdims=True))
    o_ref[0] = _norm_rows(a * _LAMBDA + att * (1.0 - _LAMBDA))


@jax.jit
def kernel(feats, node):
    del node
    b, n, d = feats.shape
    rb = 1024
    out = pl.pallas_call(
        functools.partial(_gam_kernel, rb=rb),
        grid=(b, n // rb),
        in_specs=[pl.BlockSpec((1, n, d), lambda i, j: (i, 0, 0))],
        out_specs=pl.BlockSpec((1, rb, d), lambda i, j: (i, j, 0)),
        out_shape=jax.ShapeDtypeStruct((b, n, d), jnp.float32),
    )(feats)
    return out
